# Initial kernel scaffold; baseline (speedup 1.0000x reference)
import jax
import jax.numpy as jnp
from jax.experimental import pallas as pl


def kernel(x, edge_index, batch, embed, embed_agent, W0, b0, Ws, bs, Wl, bl):
    raise NotImplementedError("write your pallas kernel here")


# speedup vs baseline: 1.4151x; 1.4151x over previous
import jax
import jax.numpy as jnp
from jax.experimental import pallas as pl


def _final_proj_kernel(p_ref, w_ref, b_ref, o_ref):
    o_ref[...] = p_ref[...] @ w_ref[...] + b_ref[...]


def _gcn_conv(h, W, b, src, dst, norm_e, norm_self):
    h = h @ W
    msg = h[src] * norm_e[:, None]
    out = jnp.zeros_like(h).at[dst].add(msg)
    out = out + h * norm_self[:, None]
    return out + b


def kernel(x, edge_index, batch, embed, embed_agent, W0, b0, Ws, bs, Wl, bl):
    NG = 64
    NL = 15
    src = edge_index[0]
    dst = edge_index[1]
    n = x.shape[0]
    div_term = 1.0 / 10000.0
    glyphs = embed[x[:, 0]]
    is_agent = embed_agent[x[:, 3]]
    xp = x[:, 1:2].astype(jnp.float32) * div_term
    yp = x[:, 2:3].astype(jnp.float32) * div_term
    xpos = jnp.concatenate([jnp.sin(xp), jnp.cos(xp)], axis=0).reshape(-1, 2)
    ypos = jnp.concatenate([jnp.sin(yp), jnp.cos(yp)], axis=0).reshape(-1, 2)
    h = jnp.concatenate([glyphs, xpos, ypos, is_agent], axis=-1)

    deg = jnp.ones((n,), jnp.float32).at[dst].add(1.0)
    dinv = jax.lax.rsqrt(deg)
    norm_e = dinv[src] * dinv[dst]
    norm_self = dinv * dinv

    h = _gcn_conv(h, W0, b0, src, dst, norm_e, norm_self)
    for i in range(NL - 1):
        h = jax.nn.relu(h)
        h = _gcn_conv(h, Ws[i], bs[i], src, dst, norm_e, norm_self)

    sums = jax.ops.segment_sum(h, batch, num_segments=NG)
    cnt = jax.ops.segment_sum(jnp.ones((n, 1), h.dtype), batch, num_segments=NG)
    pooled = sums / jnp.maximum(cnt, 1.0)

    return pl.pallas_call(
        _final_proj_kernel,
        out_shape=jax.ShapeDtypeStruct((NG, Wl.shape[1]), jnp.float32),
    )(pooled, Wl, bl)


# speedup vs baseline: 16.7992x; 11.8718x over previous
import functools

import jax
import jax.numpy as jnp
from jax import lax
from jax.experimental import pallas as pl
from jax.experimental.pallas import tpu as pltpu
from jax.experimental.pallas import tpu_sc as plsc

N = 50000
E = 1600000
H = 16
NG = 64
NL = 15

NC = 2
NS = 16
NW = NC * NS
EBW = 128
EB = 12512
EPAD = EB * EBW
BPW = EB // NW
NPAD = 50048
RPS = NPAD // NS

_mesh = plsc.VectorSubcoreMesh(core_axis_name="c", subcore_axis_name="s")


@functools.partial(
    pl.kernel,
    out_type=jax.ShapeDtypeStruct((NC, NPAD, H), jnp.float32),
    mesh=_mesh,
    scratch_types=[
        pltpu.VMEM((2, EBW), jnp.int32),
        pltpu.VMEM((EBW, H), jnp.float32),
        pltpu.VMEM_SHARED((NPAD, H), jnp.float32),
        pltpu.SemaphoreType.DMA,
    ],
    compiler_params=pltpu.CompilerParams(use_tc_tiling_on_sc=False),
)
def _spmm(v_hbm, zero_hbm, ei_hbm, out_hbm, idx_v, buf, acc, sem):
    cid = lax.axis_index("c")
    sid = lax.axis_index("s")
    wid = sid * NC + cid
    rows = pl.ds(sid * RPS, RPS)

    @pl.when(cid == 0)
    def _():
        pltpu.sync_copy(v_hbm.at[rows], acc.at[rows])

    @pl.when(cid == 1)
    def _():
        pltpu.sync_copy(zero_hbm.at[rows], acc.at[rows])

    plsc.subcore_barrier()

    @pl.loop(0, BPW)
    def _(b):
        blk = wid * BPW + b
        pltpu.sync_copy(ei_hbm.at[blk], idx_v)
        pltpu.async_copy(v_hbm.at[idx_v.at[0]], buf, sem).wait()
        pltpu.sync_copy(buf, acc.at[idx_v.at[1]], add=True)

    plsc.subcore_barrier()
    pltpu.sync_copy(acc.at[rows], out_hbm.at[cid].at[rows])


def _prep_body(x_ref, pos_ref, pd0_ref, pd1_ref, tab_ref, w0p_ref,
               dinv_ref, v_ref):
    dinv = lax.rsqrt(pd0_ref[...] + pd1_ref[...])
    dinv_ref[...] = dinv
    x0 = x_ref[:, 0:1]
    x3 = x_ref[:, 3:4]
    g = jnp.where(x0 == 0, tab_ref[0:1, :], tab_ref[1:2, :])
    g = g + jnp.where(x3 == 0, tab_ref[2:3, :], tab_ref[3:4, :])
    v_ref[...] = dinv * (g + pos_ref[...] @ w0p_ref[...])


def _layer_body(p0_ref, p1_ref, dinv_ref, b_ref, w_ref, o_ref):
    h = dinv_ref[...] * (p0_ref[...] + p1_ref[...]) + b_ref[...]
    o_ref[...] = dinv_ref[...] * (jnp.maximum(h, 0.0) @ w_ref[...])


def _final_body(p0_ref, p1_ref, dinv_ref, batch_ref, b_ref, wl_ref, bl_ref,
                o_ref, scr):
    i = pl.program_id(0)

    @pl.when(i == 0)
    def _():
        scr[...] = jnp.zeros_like(scr)

    h = dinv_ref[...] * (p0_ref[...] + p1_ref[...])
    nb = h.shape[0]
    hh = jnp.concatenate(
        [h, jnp.ones((nb, 1), jnp.float32), jnp.zeros((nb, 15), jnp.float32)],
        axis=1)
    seg = batch_ref[0, 0, :]
    onehot = (seg[:, None] == lax.broadcasted_iota(jnp.int32, (nb, NG), 1)
              ).astype(jnp.float32)
    scr[...] += lax.dot_general(onehot, hh, (((0,), (0,)), ((), ())))

    @pl.when(i == pl.num_programs(0) - 1)
    def _():
        pooled = scr[:, 0:H] / jnp.maximum(scr[:, H:H + 1], 1.0) + b_ref[...]
        o_ref[...] = pooled @ wl_ref[...] + bl_ref[...]


_BR = 3128
_FBR = 2000


def kernel(x, edge_index, batch, embed, embed_agent, W0, b0, Ws, bs, Wl, bl):
    f32 = jnp.float32

    ei = jnp.concatenate(
        [edge_index.astype(jnp.int32),
         jnp.full((2, EPAD - E), N, jnp.int32)], axis=1)
    ei_blk = jnp.stack([ei[0].reshape(EB, EBW), ei[1].reshape(EB, EBW)],
                       axis=1)

    ones_v = jnp.ones((NPAD, H), f32)
    zeros_v = jnp.zeros((NPAD, H), f32)

    div_term = 1.0 / 10000.0
    xp = x[:, 1:2].astype(f32) * div_term
    yp = x[:, 2:3].astype(f32) * div_term
    xpos = jnp.concatenate([jnp.sin(xp), jnp.cos(xp)], axis=0).reshape(-1, 2)
    ypos = jnp.concatenate([jnp.sin(yp), jnp.cos(yp)], axis=0).reshape(-1, 2)
    pos4 = jnp.concatenate([xpos, ypos], axis=1)
    pos4 = jnp.concatenate([pos4, jnp.zeros((NPAD - N, 4), f32)], axis=0)
    x_pad = jnp.concatenate(
        [x.astype(jnp.int32), jnp.zeros((NPAD - N, 4), jnp.int32)], axis=0)

    tab = jnp.concatenate([embed[0:2] @ W0[0:H],
                           embed_agent @ W0[H + 4:H + 8]], axis=0)
    w0p = W0[H:H + 4]

    pd = _spmm(ones_v, zeros_v, ei_blk)

    grid16 = (NPAD // _BR,)
    row_spec = pl.BlockSpec((_BR, H), lambda i: (i, 0))
    full44 = pl.BlockSpec((4, H), lambda i: (0, 0))
    dinv, v = pl.pallas_call(
        _prep_body,
        grid=grid16,
        in_specs=[
            pl.BlockSpec((_BR, 4), lambda i: (i, 0)),
            pl.BlockSpec((_BR, 4), lambda i: (i, 0)),
            row_spec, row_spec, full44, full44,
        ],
        out_specs=[row_spec, row_spec],
        out_shape=[jax.ShapeDtypeStruct((NPAD, H), f32),
                   jax.ShapeDtypeStruct((NPAD, H), f32)],
    )(x_pad, pos4, pd[0], pd[1], tab, w0p)

    layer_call = pl.pallas_call(
        _layer_body,
        grid=grid16,
        in_specs=[
            row_spec, row_spec, row_spec,
            pl.BlockSpec((1, H), lambda i: (0, 0)),
            pl.BlockSpec((H, H), lambda i: (0, 0)),
        ],
        out_specs=row_spec,
        out_shape=jax.ShapeDtypeStruct((NPAD, H), f32),
    )
    biases = [b0] + [bs[j] for j in range(NL - 2)]
    for i in range(NL - 1):
        p = _spmm(v, zeros_v, ei_blk)
        v = layer_call(p[0], p[1], dinv, biases[i].reshape(1, H), Ws[i])

    p = _spmm(v, zeros_v, ei_blk)
    batch_blk = batch.astype(jnp.int32).reshape(N // _FBR, 1, _FBR)
    frow = pl.BlockSpec((_FBR, H), lambda i: (i, 0))
    out = pl.pallas_call(
        _final_body,
        grid=(N // _FBR,),
        in_specs=[
            frow, frow, frow,
            pl.BlockSpec((1, 1, _FBR), lambda i: (i, 0, 0)),
            pl.BlockSpec((1, H), lambda i: (0, 0)),
            pl.BlockSpec((H, NG // 8), lambda i: (0, 0)),
            pl.BlockSpec((1, NG // 8), lambda i: (0, 0)),
        ],
        out_specs=pl.BlockSpec((NG, NG // 8), lambda i: (0, 0)),
        out_shape=jax.ShapeDtypeStruct((NG, NG // 8), f32),
        scratch_shapes=[pltpu.VMEM((NG, 2 * H), f32)],
    )(p[0][:N], p[1][:N], dinv[:N], batch_blk,
      bs[NL - 2].reshape(1, H), Wl, bl.reshape(1, NG // 8))
    return out


# speedup vs baseline: 27.6358x; 1.6451x over previous
import functools

import jax
import jax.numpy as jnp
from jax import lax
from jax.experimental import pallas as pl
from jax.experimental.pallas import tpu as pltpu
from jax.experimental.pallas import tpu_sc as plsc

N = 50000
E = 1600000
H = 16
NG = 64
NL = 15

NC = 2
NS = 16
NW = NC * NS
EBW = 128
EB = 12512
EPAD = EB * EBW
BPW = EB // NW
NPAD = 50048
RPS = NPAD // NS

_mesh = plsc.VectorSubcoreMesh(core_axis_name="c", subcore_axis_name="s")


@functools.partial(
    pl.kernel,
    out_type=jax.ShapeDtypeStruct((NC, NPAD, H), jnp.float32),
    mesh=_mesh,
    scratch_types=[
        pltpu.VMEM((2, EBW), jnp.int32),
        pltpu.VMEM((2, EBW), jnp.int32),
        pltpu.VMEM((EBW, H), jnp.float32),
        pltpu.VMEM((EBW, H), jnp.float32),
        pltpu.VMEM_SHARED((NPAD, H), jnp.float32),
        pltpu.SemaphoreType.DMA,
        pltpu.SemaphoreType.DMA,
    ],
    compiler_params=pltpu.CompilerParams(use_tc_tiling_on_sc=False),
)
def _spmm(v_hbm, zero_hbm, ei_hbm, out_hbm, ibuf0, ibuf1, gbuf0, gbuf1,
          acc, gsem0, gsem1):
    cid = lax.axis_index("c")
    sid = lax.axis_index("s")
    wid = sid * NC + cid
    rows = pl.ds(sid * RPS, RPS)

    @pl.when(cid == 0)
    def _():
        pltpu.sync_copy(v_hbm.at[rows], acc.at[rows])

    @pl.when(cid == 1)
    def _():
        pltpu.sync_copy(zero_hbm.at[rows], acc.at[rows])

    plsc.subcore_barrier()

    base = wid * BPW
    pltpu.sync_copy(ei_hbm.at[base], ibuf0)
    pltpu.make_async_copy(v_hbm.at[ibuf0.at[0]], gbuf0, gsem0).start()

    @pl.loop(0, BPW - 1, step=2)
    def _(b):
        pltpu.sync_copy(ei_hbm.at[base + b + 1], ibuf1)
        pltpu.make_async_copy(v_hbm.at[ibuf1.at[0]], gbuf1, gsem1).start()
        pltpu.make_async_copy(v_hbm.at[ibuf0.at[0]], gbuf0, gsem0).wait()
        pltpu.sync_copy(gbuf0, acc.at[ibuf0.at[1]], add=True)
        pltpu.sync_copy(ei_hbm.at[base + b + 2], ibuf0)
        pltpu.make_async_copy(v_hbm.at[ibuf0.at[0]], gbuf0, gsem0).start()
        pltpu.make_async_copy(v_hbm.at[ibuf1.at[0]], gbuf1, gsem1).wait()
        pltpu.sync_copy(gbuf1, acc.at[ibuf1.at[1]], add=True)

    pltpu.make_async_copy(v_hbm.at[ibuf0.at[0]], gbuf0, gsem0).wait()
    pltpu.sync_copy(gbuf0, acc.at[ibuf0.at[1]], add=True)

    plsc.subcore_barrier()
    pltpu.sync_copy(acc.at[rows], out_hbm.at[cid].at[rows])


def _prep_body(x_ref, pos_ref, pd0_ref, pd1_ref, tab_ref, w0p_ref,
               dinv_ref, v_ref):
    dinv = lax.rsqrt(pd0_ref[...] + pd1_ref[...])
    dinv_ref[...] = dinv
    x0 = x_ref[:, 0:1]
    x3 = x_ref[:, 3:4]
    g = jnp.where(x0 == 0, tab_ref[0:1, :], tab_ref[1:2, :])
    g = g + jnp.where(x3 == 0, tab_ref[2:3, :], tab_ref[3:4, :])
    v_ref[...] = dinv * (g + pos_ref[...] @ w0p_ref[...])


def _layer_body(p0_ref, p1_ref, dinv_ref, b_ref, w_ref, o_ref):
    h = dinv_ref[...] * (p0_ref[...] + p1_ref[...]) + b_ref[...]
    o_ref[...] = dinv_ref[...] * (jnp.maximum(h, 0.0) @ w_ref[...])


def _final_body(p0_ref, p1_ref, dinv_ref, batch_ref, b_ref, wl_ref, bl_ref,
                o_ref, scr):
    i = pl.program_id(0)

    @pl.when(i == 0)
    def _():
        scr[...] = jnp.zeros_like(scr)

    h = dinv_ref[...] * (p0_ref[...] + p1_ref[...])
    nb = h.shape[0]
    hh = jnp.concatenate(
        [h, jnp.ones((nb, 1), jnp.float32), jnp.zeros((nb, 15), jnp.float32)],
        axis=1)
    seg = batch_ref[0, 0, :]
    onehot = (seg[:, None] == lax.broadcasted_iota(jnp.int32, (nb, NG), 1)
              ).astype(jnp.float32)
    scr[...] += lax.dot_general(onehot, hh, (((0,), (0,)), ((), ())))

    @pl.when(i == pl.num_programs(0) - 1)
    def _():
        pooled = scr[:, 0:H] / jnp.maximum(scr[:, H:H + 1], 1.0) + b_ref[...]
        o_ref[...] = pooled @ wl_ref[...] + bl_ref[...]


_BR = 3128
_FBR = 2000


def kernel(x, edge_index, batch, embed, embed_agent, W0, b0, Ws, bs, Wl, bl):
    f32 = jnp.float32

    ei = jnp.concatenate(
        [edge_index.astype(jnp.int32),
         jnp.full((2, EPAD - E), N, jnp.int32)], axis=1)
    ei_blk = jnp.stack([ei[0].reshape(EB, EBW), ei[1].reshape(EB, EBW)],
                       axis=1)

    ones_v = jnp.ones((NPAD, H), f32)
    zeros_v = jnp.zeros((NPAD, H), f32)

    div_term = 1.0 / 10000.0
    xp = x[:, 1:2].astype(f32) * div_term
    yp = x[:, 2:3].astype(f32) * div_term
    xpos = jnp.concatenate([jnp.sin(xp), jnp.cos(xp)], axis=0).reshape(-1, 2)
    ypos = jnp.concatenate([jnp.sin(yp), jnp.cos(yp)], axis=0).reshape(-1, 2)
    pos4 = jnp.concatenate([xpos, ypos], axis=1)
    pos4 = jnp.concatenate([pos4, jnp.zeros((NPAD - N, 4), f32)], axis=0)
    x_pad = jnp.concatenate(
        [x.astype(jnp.int32), jnp.zeros((NPAD - N, 4), jnp.int32)], axis=0)

    tab = jnp.concatenate([embed[0:2] @ W0[0:H],
                           embed_agent @ W0[H + 4:H + 8]], axis=0)
    w0p = W0[H:H + 4]

    pd = _spmm(ones_v, zeros_v, ei_blk)

    grid16 = (NPAD // _BR,)
    row_spec = pl.BlockSpec((_BR, H), lambda i: (i, 0))
    full44 = pl.BlockSpec((4, H), lambda i: (0, 0))
    dinv, v = pl.pallas_call(
        _prep_body,
        grid=grid16,
        in_specs=[
            pl.BlockSpec((_BR, 4), lambda i: (i, 0)),
            pl.BlockSpec((_BR, 4), lambda i: (i, 0)),
            row_spec, row_spec, full44, full44,
        ],
        out_specs=[row_spec, row_spec],
        out_shape=[jax.ShapeDtypeStruct((NPAD, H), f32),
                   jax.ShapeDtypeStruct((NPAD, H), f32)],
    )(x_pad, pos4, pd[0], pd[1], tab, w0p)

    layer_call = pl.pallas_call(
        _layer_body,
        grid=grid16,
        in_specs=[
            row_spec, row_spec, row_spec,
            pl.BlockSpec((1, H), lambda i: (0, 0)),
            pl.BlockSpec((H, H), lambda i: (0, 0)),
        ],
        out_specs=row_spec,
        out_shape=jax.ShapeDtypeStruct((NPAD, H), f32),
    )
    biases = [b0] + [bs[j] for j in range(NL - 2)]
    for i in range(NL - 1):
        p = _spmm(v, zeros_v, ei_blk)
        v = layer_call(p[0], p[1], dinv, biases[i].reshape(1, H), Ws[i])

    p = _spmm(v, zeros_v, ei_blk)
    batch_blk = batch.astype(jnp.int32).reshape(N // _FBR, 1, _FBR)
    frow = pl.BlockSpec((_FBR, H), lambda i: (i, 0))
    out = pl.pallas_call(
        _final_body,
        grid=(N // _FBR,),
        in_specs=[
            frow, frow, frow,
            pl.BlockSpec((1, 1, _FBR), lambda i: (i, 0, 0)),
            pl.BlockSpec((1, H), lambda i: (0, 0)),
            pl.BlockSpec((H, NG // 8), lambda i: (0, 0)),
            pl.BlockSpec((1, NG // 8), lambda i: (0, 0)),
        ],
        out_specs=pl.BlockSpec((NG, NG // 8), lambda i: (0, 0)),
        out_shape=jax.ShapeDtypeStruct((NG, NG // 8), f32),
        scratch_shapes=[pltpu.VMEM((NG, 2 * H), f32)],
    )(p[0][:N], p[1][:N], dinv[:N], batch_blk,
      bs[NL - 2].reshape(1, H), Wl, bl.reshape(1, NG // 8))
    return out


# speedup vs baseline: 35.8881x; 1.2986x over previous
import functools

import jax
import jax.numpy as jnp
from jax import lax
from jax.experimental import pallas as pl
from jax.experimental.pallas import tpu as pltpu
from jax.experimental.pallas import tpu_sc as plsc

N = 50000
E = 1600000
H = 16
NG = 64
NL = 15

NC = 2
NS = 16
NW = NC * NS
EBW = 128
EB = 12544
EPAD = EB * EBW
BPW = EB // NW
NPAD = 50048
RPS = NPAD // NS

_mesh = plsc.VectorSubcoreMesh(core_axis_name="c", subcore_axis_name="s")


@functools.partial(
    pl.kernel,
    out_type=jax.ShapeDtypeStruct((NC, NPAD, H), jnp.float32),
    mesh=_mesh,
    scratch_types=[
        pltpu.VMEM((BPW // 4, 2, EBW), jnp.int32),
        pltpu.VMEM((BPW // 4, 2, EBW), jnp.int32),
        pltpu.VMEM((EBW, H), jnp.float32),
        pltpu.VMEM((EBW, H), jnp.float32),
        pltpu.VMEM_SHARED((NPAD, H), jnp.float32),
        pltpu.SemaphoreType.DMA,
        pltpu.SemaphoreType.DMA,
        pltpu.SemaphoreType.DMA,
        pltpu.SemaphoreType.DMA,
    ],
    compiler_params=pltpu.CompilerParams(use_tc_tiling_on_sc=False),
)
def _spmm(v_hbm, zero_hbm, ei_hbm, out_hbm, idx0, idx1, gbuf0, gbuf1,
          acc, gsem0, gsem1, isem0, isem1):
    cid = lax.axis_index("c")
    sid = lax.axis_index("s")
    wid = sid * NC + cid
    rows = pl.ds(sid * RPS, RPS)
    base = wid * BPW
    qbpw = BPW // 4

    pltpu.sync_copy(ei_hbm.at[pl.ds(base, qbpw)], idx0)
    pltpu.make_async_copy(ei_hbm.at[pl.ds(base + qbpw, qbpw)], idx1,
                          isem1).start()

    @pl.when(cid == 0)
    def _():
        pltpu.sync_copy(v_hbm.at[rows], acc.at[rows])

    @pl.when(cid == 1)
    def _():
        pltpu.sync_copy(zero_hbm.at[rows], acc.at[rows])

    plsc.subcore_barrier()

    def _run_chunk(idx):
        pltpu.make_async_copy(v_hbm.at[idx.at[0].at[0]], gbuf0,
                              gsem0).start()

        @pl.loop(0, qbpw - 2, step=2)
        def _(b):
            pltpu.make_async_copy(v_hbm.at[idx.at[b + 1].at[0]], gbuf1,
                                  gsem1).start()
            pltpu.make_async_copy(v_hbm.at[idx.at[b].at[0]], gbuf0,
                                  gsem0).wait()
            pltpu.sync_copy(gbuf0, acc.at[idx.at[b].at[1]], add=True)
            pltpu.make_async_copy(v_hbm.at[idx.at[b + 2].at[0]], gbuf0,
                                  gsem0).start()
            pltpu.make_async_copy(v_hbm.at[idx.at[b + 1].at[0]], gbuf1,
                                  gsem1).wait()
            pltpu.sync_copy(gbuf1, acc.at[idx.at[b + 1].at[1]], add=True)

        pltpu.make_async_copy(v_hbm.at[idx.at[qbpw - 1].at[0]], gbuf1,
                              gsem1).start()
        pltpu.make_async_copy(v_hbm.at[idx.at[qbpw - 2].at[0]], gbuf0,
                              gsem0).wait()
        pltpu.sync_copy(gbuf0, acc.at[idx.at[qbpw - 2].at[1]], add=True)
        pltpu.make_async_copy(v_hbm.at[idx.at[qbpw - 1].at[0]], gbuf1,
                              gsem1).wait()
        pltpu.sync_copy(gbuf1, acc.at[idx.at[qbpw - 1].at[1]], add=True)

    _run_chunk(idx0)
    pltpu.make_async_copy(ei_hbm.at[pl.ds(base + 2 * qbpw, qbpw)], idx0,
                          isem0).start()
    pltpu.make_async_copy(ei_hbm.at[pl.ds(base + qbpw, qbpw)], idx1,
                          isem1).wait()
    _run_chunk(idx1)
    pltpu.make_async_copy(ei_hbm.at[pl.ds(base + 3 * qbpw, qbpw)], idx1,
                          isem1).start()
    pltpu.make_async_copy(ei_hbm.at[pl.ds(base + 2 * qbpw, qbpw)], idx0,
                          isem0).wait()
    _run_chunk(idx0)
    pltpu.make_async_copy(ei_hbm.at[pl.ds(base + 3 * qbpw, qbpw)], idx1,
                          isem1).wait()
    _run_chunk(idx1)

    plsc.subcore_barrier()
    pltpu.sync_copy(acc.at[rows], out_hbm.at[cid].at[rows])


def _prep_body(x_ref, pos_ref, pd0_ref, pd1_ref, tab_ref, w0p_ref,
               dinv_ref, v_ref):
    dinv = lax.rsqrt(pd0_ref[...] + pd1_ref[...])
    dinv_ref[...] = dinv
    x0 = x_ref[:, 0:1]
    x3 = x_ref[:, 3:4]
    g = jnp.where(x0 == 0, tab_ref[0:1, :], tab_ref[1:2, :])
    g = g + jnp.where(x3 == 0, tab_ref[2:3, :], tab_ref[3:4, :])
    v_ref[...] = dinv * (g + pos_ref[...] @ w0p_ref[...])


def _layer_body(p0_ref, p1_ref, dinv_ref, b_ref, w_ref, o_ref):
    h = dinv_ref[...] * (p0_ref[...] + p1_ref[...]) + b_ref[...]
    o_ref[...] = dinv_ref[...] * (jnp.maximum(h, 0.0) @ w_ref[...])


def _final_body(p0_ref, p1_ref, dinv_ref, batch_ref, b_ref, wl_ref, bl_ref,
                o_ref, scr):
    i = pl.program_id(0)

    @pl.when(i == 0)
    def _():
        scr[...] = jnp.zeros_like(scr)

    h = dinv_ref[...] * (p0_ref[...] + p1_ref[...])
    nb = h.shape[0]
    hh = jnp.concatenate(
        [h, jnp.ones((nb, 1), jnp.float32), jnp.zeros((nb, 15), jnp.float32)],
        axis=1)
    seg = batch_ref[0, 0, :]
    onehot = (seg[:, None] == lax.broadcasted_iota(jnp.int32, (nb, NG), 1)
              ).astype(jnp.float32)
    scr[...] += lax.dot_general(onehot, hh, (((0,), (0,)), ((), ())))

    @pl.when(i == pl.num_programs(0) - 1)
    def _():
        pooled = scr[:, 0:H] / jnp.maximum(scr[:, H:H + 1], 1.0) + b_ref[...]
        o_ref[...] = pooled @ wl_ref[...] + bl_ref[...]


_BR = 3128
_FBR = 2000


def kernel(x, edge_index, batch, embed, embed_agent, W0, b0, Ws, bs, Wl, bl):
    f32 = jnp.float32

    pad_idx = N + (jnp.arange(EPAD - E, dtype=jnp.int32) % (NPAD - N))
    ei = jnp.concatenate(
        [edge_index.astype(jnp.int32),
         jnp.stack([pad_idx, pad_idx])], axis=1)
    ei_blk = jnp.stack([ei[0].reshape(EB, EBW), ei[1].reshape(EB, EBW)],
                       axis=1)

    ones_v = jnp.ones((NPAD, H), f32)
    zeros_v = jnp.zeros((NPAD, H), f32)

    div_term = 1.0 / 10000.0
    xp = x[:, 1:2].astype(f32) * div_term
    yp = x[:, 2:3].astype(f32) * div_term
    xpos = jnp.concatenate([jnp.sin(xp), jnp.cos(xp)], axis=0).reshape(-1, 2)
    ypos = jnp.concatenate([jnp.sin(yp), jnp.cos(yp)], axis=0).reshape(-1, 2)
    pos4 = jnp.concatenate([xpos, ypos], axis=1)
    pos4 = jnp.concatenate([pos4, jnp.zeros((NPAD - N, 4), f32)], axis=0)
    x_pad = jnp.concatenate(
        [x.astype(jnp.int32), jnp.zeros((NPAD - N, 4), jnp.int32)], axis=0)

    tab = jnp.concatenate([embed[0:2] @ W0[0:H],
                           embed_agent @ W0[H + 4:H + 8]], axis=0)
    w0p = W0[H:H + 4]

    pd = _spmm(ones_v, zeros_v, ei_blk)

    grid16 = (NPAD // _BR,)
    row_spec = pl.BlockSpec((_BR, H), lambda i: (i, 0))
    full44 = pl.BlockSpec((4, H), lambda i: (0, 0))
    dinv, v = pl.pallas_call(
        _prep_body,
        grid=grid16,
        in_specs=[
            pl.BlockSpec((_BR, 4), lambda i: (i, 0)),
            pl.BlockSpec((_BR, 4), lambda i: (i, 0)),
            row_spec, row_spec, full44, full44,
        ],
        out_specs=[row_spec, row_spec],
        out_shape=[jax.ShapeDtypeStruct((NPAD, H), f32),
                   jax.ShapeDtypeStruct((NPAD, H), f32)],
    )(x_pad, pos4, pd[0], pd[1], tab, w0p)

    layer_call = pl.pallas_call(
        _layer_body,
        grid=grid16,
        in_specs=[
            row_spec, row_spec, row_spec,
            pl.BlockSpec((1, H), lambda i: (0, 0)),
            pl.BlockSpec((H, H), lambda i: (0, 0)),
        ],
        out_specs=row_spec,
        out_shape=jax.ShapeDtypeStruct((NPAD, H), f32),
    )
    biases = [b0] + [bs[j] for j in range(NL - 2)]
    for i in range(NL - 1):
        p = _spmm(v, zeros_v, ei_blk)
        v = layer_call(p[0], p[1], dinv, biases[i].reshape(1, H), Ws[i])

    p = _spmm(v, zeros_v, ei_blk)
    batch_blk = batch.astype(jnp.int32).reshape(N // _FBR, 1, _FBR)
    frow = pl.BlockSpec((_FBR, H), lambda i: (i, 0))
    out = pl.pallas_call(
        _final_body,
        grid=(N // _FBR,),
        in_specs=[
            frow, frow, frow,
            pl.BlockSpec((1, 1, _FBR), lambda i: (i, 0, 0)),
            pl.BlockSpec((1, H), lambda i: (0, 0)),
            pl.BlockSpec((H, NG // 8), lambda i: (0, 0)),
            pl.BlockSpec((1, NG // 8), lambda i: (0, 0)),
        ],
        out_specs=pl.BlockSpec((NG, NG // 8), lambda i: (0, 0)),
        out_shape=jax.ShapeDtypeStruct((NG, NG // 8), f32),
        scratch_shapes=[pltpu.VMEM((NG, 2 * H), f32)],
    )(p[0][:N], p[1][:N], dinv[:N], batch_blk,
      bs[NL - 2].reshape(1, H), Wl, bl.reshape(1, NG // 8))
    return out


# speedup vs baseline: 38.8777x; 1.0833x over previous
import functools

import jax
import jax.numpy as jnp
from jax import lax
from jax.experimental import pallas as pl
from jax.experimental.pallas import tpu as pltpu
from jax.experimental.pallas import tpu_sc as plsc

N = 50000
E = 1600000
H = 16
NG = 64
NL = 15

NC = 2
NS = 16
NW = NC * NS
EBW = 128
EB = 12544
EPAD = EB * EBW
BPW = EB // NW
NPAD = 50048
RPS = NPAD // NS

_mesh = plsc.VectorSubcoreMesh(core_axis_name="c", subcore_axis_name="s")


@functools.partial(
    pl.kernel,
    out_type=jax.ShapeDtypeStruct((NC, NPAD, H), jnp.float32),
    mesh=_mesh,
    scratch_types=[
        pltpu.VMEM((BPW // 4, 2, EBW), jnp.int32),
        pltpu.VMEM((BPW // 4, 2, EBW), jnp.int32),
        pltpu.VMEM((EBW, H), jnp.float32),
        pltpu.VMEM((EBW, H), jnp.float32),
        pltpu.VMEM((EBW, H), jnp.float32),
        pltpu.VMEM((EBW, H), jnp.float32),
        pltpu.VMEM_SHARED((NPAD, H), jnp.float32),
        pltpu.SemaphoreType.DMA,
        pltpu.SemaphoreType.DMA,
        pltpu.SemaphoreType.DMA,
        pltpu.SemaphoreType.DMA,
        pltpu.SemaphoreType.DMA,
        pltpu.SemaphoreType.DMA,
        pltpu.SemaphoreType.DMA,
        pltpu.SemaphoreType.DMA,
        pltpu.SemaphoreType.DMA,
        pltpu.SemaphoreType.DMA,
    ],
    compiler_params=pltpu.CompilerParams(use_tc_tiling_on_sc=False),
)
def _spmm(v_hbm, zero_hbm, ei_hbm, out_hbm, idx0, idx1,
          gbuf0, gbuf1, gbuf2, gbuf3, acc,
          gsem0, gsem1, gsem2, gsem3,
          ssem0, ssem1, ssem2, ssem3, isem0, isem1):
    cid = lax.axis_index("c")
    sid = lax.axis_index("s")
    wid = sid * NC + cid
    rows = pl.ds(sid * RPS, RPS)
    base = wid * BPW
    qbpw = BPW // 4

    pltpu.sync_copy(ei_hbm.at[pl.ds(base, qbpw)], idx0)
    pltpu.make_async_copy(ei_hbm.at[pl.ds(base + qbpw, qbpw)], idx1,
                          isem1).start()

    @pl.when(cid == 0)
    def _():
        pltpu.sync_copy(v_hbm.at[rows], acc.at[rows])

    @pl.when(cid == 1)
    def _():
        pltpu.sync_copy(zero_hbm.at[rows], acc.at[rows])

    plsc.subcore_barrier()

    gbufs = (gbuf0, gbuf1, gbuf2, gbuf3)
    gsems = (gsem0, gsem1, gsem2, gsem3)
    ssems = (ssem0, ssem1, ssem2, ssem3)

    def _run_chunk(idx):
        def sg(b):
            r = b % 4
            pltpu.make_async_copy(v_hbm.at[idx.at[b].at[0]], gbufs[r],
                                  gsems[r]).start()

        sg(0)
        sg(1)
        pltpu.make_async_copy(v_hbm.at[idx.at[0].at[0]], gbuf0,
                              gsems[0]).wait()
        pltpu.make_async_copy(gbuf0, acc.at[idx.at[0].at[1]],
                              ssems[0]).start(add=True)
        sg(2)
        pltpu.make_async_copy(v_hbm.at[idx.at[1].at[0]], gbuf1,
                              gsems[1]).wait()
        pltpu.make_async_copy(gbuf1, acc.at[idx.at[1].at[1]],
                              ssems[1]).start(add=True)
        sg(3)

        @pl.loop(2, qbpw - 4, step=4)
        def _(b):
            for k in range(4):
                r = (2 + k) % 4
                bb = b + k
                pltpu.make_async_copy(v_hbm.at[idx.at[bb].at[0]], gbufs[r],
                                      gsems[r]).wait()
                pltpu.make_async_copy(gbufs[r], acc.at[idx.at[bb].at[1]],
                                      ssems[r]).start(add=True)
                r2 = (r + 2) % 4
                pltpu.make_async_copy(gbufs[r2], acc.at[idx.at[bb].at[1]],
                                      ssems[r2]).wait()
                pltpu.make_async_copy(v_hbm.at[idx.at[bb + 2].at[0]],
                                      gbufs[r2], gsems[r2]).start()

        for bb in (qbpw - 4, qbpw - 3):
            r = bb % 4
            pltpu.make_async_copy(v_hbm.at[idx.at[bb].at[0]], gbufs[r],
                                  gsems[r]).wait()
            pltpu.make_async_copy(gbufs[r], acc.at[idx.at[bb].at[1]],
                                  ssems[r]).start(add=True)
            r2 = (r + 2) % 4
            pltpu.make_async_copy(gbufs[r2], acc.at[idx.at[bb].at[1]],
                                  ssems[r2]).wait()
            pltpu.make_async_copy(v_hbm.at[idx.at[bb + 2].at[0]],
                                  gbufs[r2], gsems[r2]).start()
        for bb in (qbpw - 2, qbpw - 1):
            r = bb % 4
            pltpu.make_async_copy(v_hbm.at[idx.at[bb].at[0]], gbufs[r],
                                  gsems[r]).wait()
            pltpu.make_async_copy(gbufs[r], acc.at[idx.at[bb].at[1]],
                                  ssems[r]).start(add=True)
        for r in range(4):
            pltpu.make_async_copy(gbufs[r], acc.at[idx.at[0].at[1]],
                                  ssems[r]).wait()

    _run_chunk(idx0)
    pltpu.make_async_copy(ei_hbm.at[pl.ds(base + 2 * qbpw, qbpw)], idx0,
                          isem0).start()
    pltpu.make_async_copy(ei_hbm.at[pl.ds(base + qbpw, qbpw)], idx1,
                          isem1).wait()
    _run_chunk(idx1)
    pltpu.make_async_copy(ei_hbm.at[pl.ds(base + 3 * qbpw, qbpw)], idx1,
                          isem1).start()
    pltpu.make_async_copy(ei_hbm.at[pl.ds(base + 2 * qbpw, qbpw)], idx0,
                          isem0).wait()
    _run_chunk(idx0)
    pltpu.make_async_copy(ei_hbm.at[pl.ds(base + 3 * qbpw, qbpw)], idx1,
                          isem1).wait()
    _run_chunk(idx1)

    plsc.subcore_barrier()
    pltpu.sync_copy(acc.at[rows], out_hbm.at[cid].at[rows])


def _prep_body(x_ref, pos_ref, pd0_ref, pd1_ref, tab_ref, w0p_ref,
               dinv_ref, v_ref):
    dinv = lax.rsqrt(pd0_ref[...] + pd1_ref[...])
    dinv_ref[...] = dinv
    x0 = x_ref[:, 0:1]
    x3 = x_ref[:, 3:4]
    g = jnp.where(x0 == 0, tab_ref[0:1, :], tab_ref[1:2, :])
    g = g + jnp.where(x3 == 0, tab_ref[2:3, :], tab_ref[3:4, :])
    v_ref[...] = dinv * (g + pos_ref[...] @ w0p_ref[...])


def _layer_body(p0_ref, p1_ref, dinv_ref, b_ref, w_ref, o_ref):
    h = dinv_ref[...] * (p0_ref[...] + p1_ref[...]) + b_ref[...]
    o_ref[...] = dinv_ref[...] * (jnp.maximum(h, 0.0) @ w_ref[...])


def _final_body(p0_ref, p1_ref, dinv_ref, batch_ref, b_ref, wl_ref, bl_ref,
                o_ref, scr):
    i = pl.program_id(0)

    @pl.when(i == 0)
    def _():
        scr[...] = jnp.zeros_like(scr)

    h = dinv_ref[...] * (p0_ref[...] + p1_ref[...])
    nb = h.shape[0]
    hh = jnp.concatenate(
        [h, jnp.ones((nb, 1), jnp.float32), jnp.zeros((nb, 15), jnp.float32)],
        axis=1)
    seg = batch_ref[0, 0, :]
    onehot = (seg[:, None] == lax.broadcasted_iota(jnp.int32, (nb, NG), 1)
              ).astype(jnp.float32)
    scr[...] += lax.dot_general(onehot, hh, (((0,), (0,)), ((), ())))

    @pl.when(i == pl.num_programs(0) - 1)
    def _():
        pooled = scr[:, 0:H] / jnp.maximum(scr[:, H:H + 1], 1.0) + b_ref[...]
        o_ref[...] = pooled @ wl_ref[...] + bl_ref[...]


_BR = 3128
_FBR = 2000


def kernel(x, edge_index, batch, embed, embed_agent, W0, b0, Ws, bs, Wl, bl):
    f32 = jnp.float32

    pad_idx = N + (jnp.arange(EPAD - E, dtype=jnp.int32) % (NPAD - N))
    ei = jnp.concatenate(
        [edge_index.astype(jnp.int32),
         jnp.stack([pad_idx, pad_idx])], axis=1)
    ei_blk = jnp.stack([ei[0].reshape(EB, EBW), ei[1].reshape(EB, EBW)],
                       axis=1)

    ones_v = jnp.ones((NPAD, H), f32)
    zeros_v = jnp.zeros((NPAD, H), f32)

    div_term = 1.0 / 10000.0
    xp = x[:, 1:2].astype(f32) * div_term
    yp = x[:, 2:3].astype(f32) * div_term
    xpos = jnp.concatenate([jnp.sin(xp), jnp.cos(xp)], axis=0).reshape(-1, 2)
    ypos = jnp.concatenate([jnp.sin(yp), jnp.cos(yp)], axis=0).reshape(-1, 2)
    pos4 = jnp.concatenate([xpos, ypos], axis=1)
    pos4 = jnp.concatenate([pos4, jnp.zeros((NPAD - N, 4), f32)], axis=0)
    x_pad = jnp.concatenate(
        [x.astype(jnp.int32), jnp.zeros((NPAD - N, 4), jnp.int32)], axis=0)

    tab = jnp.concatenate([embed[0:2] @ W0[0:H],
                           embed_agent @ W0[H + 4:H + 8]], axis=0)
    w0p = W0[H:H + 4]

    pd = _spmm(ones_v, zeros_v, ei_blk)

    grid16 = (NPAD // _BR,)
    row_spec = pl.BlockSpec((_BR, H), lambda i: (i, 0))
    full44 = pl.BlockSpec((4, H), lambda i: (0, 0))
    dinv, v = pl.pallas_call(
        _prep_body,
        grid=grid16,
        in_specs=[
            pl.BlockSpec((_BR, 4), lambda i: (i, 0)),
            pl.BlockSpec((_BR, 4), lambda i: (i, 0)),
            row_spec, row_spec, full44, full44,
        ],
        out_specs=[row_spec, row_spec],
        out_shape=[jax.ShapeDtypeStruct((NPAD, H), f32),
                   jax.ShapeDtypeStruct((NPAD, H), f32)],
    )(x_pad, pos4, pd[0], pd[1], tab, w0p)

    layer_call = pl.pallas_call(
        _layer_body,
        grid=grid16,
        in_specs=[
            row_spec, row_spec, row_spec,
            pl.BlockSpec((1, H), lambda i: (0, 0)),
            pl.BlockSpec((H, H), lambda i: (0, 0)),
        ],
        out_specs=row_spec,
        out_shape=jax.ShapeDtypeStruct((NPAD, H), f32),
    )
    biases = [b0] + [bs[j] for j in range(NL - 2)]
    for i in range(NL - 1):
        p = _spmm(v, zeros_v, ei_blk)
        v = layer_call(p[0], p[1], dinv, biases[i].reshape(1, H), Ws[i])

    p = _spmm(v, zeros_v, ei_blk)
    batch_blk = batch.astype(jnp.int32).reshape(N // _FBR, 1, _FBR)
    frow = pl.BlockSpec((_FBR, H), lambda i: (i, 0))
    out = pl.pallas_call(
        _final_body,
        grid=(N // _FBR,),
        in_specs=[
            frow, frow, frow,
            pl.BlockSpec((1, 1, _FBR), lambda i: (i, 0, 0)),
            pl.BlockSpec((1, H), lambda i: (0, 0)),
            pl.BlockSpec((H, NG // 8), lambda i: (0, 0)),
            pl.BlockSpec((1, NG // 8), lambda i: (0, 0)),
        ],
        out_specs=pl.BlockSpec((NG, NG // 8), lambda i: (0, 0)),
        out_shape=jax.ShapeDtypeStruct((NG, NG // 8), f32),
        scratch_shapes=[pltpu.VMEM((NG, 2 * H), f32)],
    )(p[0][:N], p[1][:N], dinv[:N], batch_blk,
      bs[NL - 2].reshape(1, H), Wl, bl.reshape(1, NG // 8))
    return out


# speedup vs baseline: 62.0779x; 1.5967x over previous
import functools

import jax
import jax.numpy as jnp
from jax import lax
from jax.experimental import pallas as pl
from jax.experimental.pallas import tpu as pltpu
from jax.experimental.pallas import tpu_sc as plsc

N = 50000
E = 1600000
H = 16
NG = 64
NL = 15
OUT = 8

NC = 2
NS = 16
NW = NC * NS
EBW = 128
EB = 12544
EPAD = EB * EBW
BPW = EB // NW
NPAD = 50048
RPS = NPAD // NS

_mesh = plsc.VectorSubcoreMesh(core_axis_name="c", subcore_axis_name="s")


@functools.partial(
    pl.kernel,
    out_type=jax.ShapeDtypeStruct((NC, NPAD, H), jnp.float32),
    mesh=_mesh,
    scratch_types=[
        pltpu.VMEM((BPW // 4, 2, EBW), jnp.int32),
        pltpu.VMEM((BPW // 4, 2, EBW), jnp.int32),
        pltpu.VMEM((EBW, H), jnp.float32),
        pltpu.VMEM((EBW, H), jnp.float32),
        pltpu.VMEM((EBW, H), jnp.float32),
        pltpu.VMEM((EBW, H), jnp.float32),
        pltpu.VMEM_SHARED((NPAD, H), jnp.float32),
        pltpu.SemaphoreType.DMA,
        pltpu.SemaphoreType.DMA,
        pltpu.SemaphoreType.DMA,
        pltpu.SemaphoreType.DMA,
        pltpu.SemaphoreType.DMA,
        pltpu.SemaphoreType.DMA,
        pltpu.SemaphoreType.DMA,
        pltpu.SemaphoreType.DMA,
        pltpu.SemaphoreType.DMA,
        pltpu.SemaphoreType.DMA,
    ],
    compiler_params=pltpu.CompilerParams(use_tc_tiling_on_sc=False),
)
def _spmm(v_hbm, zero_hbm, ei_hbm, out_hbm, idx0, idx1,
          gbuf0, gbuf1, gbuf2, gbuf3, acc,
          gsem0, gsem1, gsem2, gsem3,
          ssem0, ssem1, ssem2, ssem3, isem0, isem1):
    cid = lax.axis_index("c")
    sid = lax.axis_index("s")
    wid = sid * NC + cid
    rows = pl.ds(sid * RPS, RPS)
    base = wid * BPW
    qbpw = BPW // 4

    pltpu.sync_copy(ei_hbm.at[pl.ds(base, qbpw)], idx0)
    pltpu.make_async_copy(ei_hbm.at[pl.ds(base + qbpw, qbpw)], idx1,
                          isem1).start()

    @pl.when(cid == 0)
    def _():
        pltpu.sync_copy(v_hbm.at[rows], acc.at[rows])

    @pl.when(cid == 1)
    def _():
        pltpu.sync_copy(zero_hbm.at[rows], acc.at[rows])

    plsc.subcore_barrier()

    gbufs = (gbuf0, gbuf1, gbuf2, gbuf3)
    gsems = (gsem0, gsem1, gsem2, gsem3)
    ssems = (ssem0, ssem1, ssem2, ssem3)

    def _run_chunk(idx):
        def sg(b):
            r = b % 4
            pltpu.make_async_copy(v_hbm.at[idx.at[b].at[0]], gbufs[r],
                                  gsems[r]).start()

        sg(0)
        sg(1)
        pltpu.make_async_copy(v_hbm.at[idx.at[0].at[0]], gbuf0,
                              gsems[0]).wait()
        pltpu.make_async_copy(gbuf0, acc.at[idx.at[0].at[1]],
                              ssems[0]).start(add=True)
        sg(2)
        pltpu.make_async_copy(v_hbm.at[idx.at[1].at[0]], gbuf1,
                              gsems[1]).wait()
        pltpu.make_async_copy(gbuf1, acc.at[idx.at[1].at[1]],
                              ssems[1]).start(add=True)
        sg(3)

        @pl.loop(2, qbpw - 4, step=4)
        def _(b):
            for k in range(4):
                r = (2 + k) % 4
                bb = b + k
                pltpu.make_async_copy(v_hbm.at[idx.at[bb].at[0]], gbufs[r],
                                      gsems[r]).wait()
                pltpu.make_async_copy(gbufs[r], acc.at[idx.at[bb].at[1]],
                                      ssems[r]).start(add=True)
                r2 = (r + 2) % 4
                pltpu.make_async_copy(gbufs[r2], acc.at[idx.at[bb].at[1]],
                                      ssems[r2]).wait()
                pltpu.make_async_copy(v_hbm.at[idx.at[bb + 2].at[0]],
                                      gbufs[r2], gsems[r2]).start()

        for bb in (qbpw - 4, qbpw - 3):
            r = bb % 4
            pltpu.make_async_copy(v_hbm.at[idx.at[bb].at[0]], gbufs[r],
                                  gsems[r]).wait()
            pltpu.make_async_copy(gbufs[r], acc.at[idx.at[bb].at[1]],
                                  ssems[r]).start(add=True)
            r2 = (r + 2) % 4
            pltpu.make_async_copy(gbufs[r2], acc.at[idx.at[bb].at[1]],
                                  ssems[r2]).wait()
            pltpu.make_async_copy(v_hbm.at[idx.at[bb + 2].at[0]],
                                  gbufs[r2], gsems[r2]).start()
        for bb in (qbpw - 2, qbpw - 1):
            r = bb % 4
            pltpu.make_async_copy(v_hbm.at[idx.at[bb].at[0]], gbufs[r],
                                  gsems[r]).wait()
            pltpu.make_async_copy(gbufs[r], acc.at[idx.at[bb].at[1]],
                                  ssems[r]).start(add=True)
        for r in range(4):
            pltpu.make_async_copy(gbufs[r], acc.at[idx.at[0].at[1]],
                                  ssems[r]).wait()

    _run_chunk(idx0)
    pltpu.make_async_copy(ei_hbm.at[pl.ds(base + 2 * qbpw, qbpw)], idx0,
                          isem0).start()
    pltpu.make_async_copy(ei_hbm.at[pl.ds(base + qbpw, qbpw)], idx1,
                          isem1).wait()
    _run_chunk(idx1)
    pltpu.make_async_copy(ei_hbm.at[pl.ds(base + 3 * qbpw, qbpw)], idx1,
                          isem1).start()
    pltpu.make_async_copy(ei_hbm.at[pl.ds(base + 2 * qbpw, qbpw)], idx0,
                          isem0).wait()
    _run_chunk(idx0)
    pltpu.make_async_copy(ei_hbm.at[pl.ds(base + 3 * qbpw, qbpw)], idx1,
                          isem1).wait()
    _run_chunk(idx1)

    plsc.subcore_barrier()
    pltpu.sync_copy(acc.at[rows], out_hbm.at[cid].at[rows])


HP = NPAD // 8


def _prep_body(pd_ref, m0_ref, m3_ref, pos_ref, tabs_ref, w0p8_ref,
               dinv_ref, v_ref):
    dinv = lax.rsqrt(pd_ref[0] + pd_ref[1])
    dinv_ref[...] = dinv
    g = (tabs_ref[0:1, :] + m0_ref[...] * tabs_ref[1:2, :]
         + m3_ref[...] * tabs_ref[2:3, :])
    v_ref[...] = dinv * (g + pos_ref[...] @ w0p8_ref[...])


def _layer_body(p_ref, dinv_ref, b_ref, w8_ref, o_ref):
    h = dinv_ref[...] * (p_ref[0] + p_ref[1]) + b_ref[...]
    o_ref[...] = dinv_ref[...] * (jnp.maximum(h, 0.0) @ w8_ref[...])


def _final_body(p_ref, dinv_ref, batch_ref, b_ref, wl_ref, bl_ref, o_ref):
    h = dinv_ref[...] * (p_ref[0] + p_ref[1])
    sums = jnp.zeros((NG, H), jnp.float32)
    cnt = jnp.zeros((NG, 1), jnp.float32)
    iota = lax.broadcasted_iota(jnp.int32, (HP, NG), 1)
    for j in range(8):
        oh = (batch_ref[:, j:j + 1] == iota).astype(jnp.float32)
        sums += lax.dot_general(oh, h[:, j * H:(j + 1) * H],
                                (((0,), (0,)), ((), ())))
        cnt += jnp.sum(oh, axis=0)[:, None]
    pooled = sums / jnp.maximum(cnt, 1.0) + b_ref[...]
    o_ref[...] = pooled @ wl_ref[...] + bl_ref[...]


def kernel(x, edge_index, batch, embed, embed_agent, W0, b0, Ws, bs, Wl, bl):
    f32 = jnp.float32

    pad_idx = N + (jnp.arange(EPAD - E, dtype=jnp.int32) % (NPAD - N))
    ei = jnp.concatenate(
        [edge_index.astype(jnp.int32),
         jnp.stack([pad_idx, pad_idx])], axis=1)
    ei_blk = jnp.stack([ei[0].reshape(EB, EBW), ei[1].reshape(EB, EBW)],
                       axis=1)

    ones_v = jnp.ones((NPAD, H), f32)
    zeros_v = jnp.zeros((NPAD, H), f32)

    div_term = 1.0 / 10000.0
    xp = x[:, 1:2].astype(f32) * div_term
    yp = x[:, 2:3].astype(f32) * div_term
    xpos = jnp.concatenate([jnp.sin(xp), jnp.cos(xp)], axis=0).reshape(-1, 2)
    ypos = jnp.concatenate([jnp.sin(yp), jnp.cos(yp)], axis=0).reshape(-1, 2)
    pos4 = jnp.concatenate([xpos, ypos], axis=1)
    pos4 = jnp.concatenate([pos4, jnp.zeros((NPAD - N, 4), f32)], axis=0)
    posP = pos4.reshape(HP, 32)
    x_pad = jnp.concatenate(
        [x.astype(jnp.int32), jnp.zeros((NPAD - N, 4), jnp.int32)], axis=0)
    m0 = jnp.repeat(x_pad[:, 0].astype(f32), H).reshape(HP, 128)
    m3 = jnp.repeat(x_pad[:, 3].astype(f32), H).reshape(HP, 128)

    e0 = embed[0] @ W0[0:H] + embed_agent[0] @ W0[H + 4:H + 8]
    de0 = (embed[1] - embed[0]) @ W0[0:H]
    da3 = (embed_agent[1] - embed_agent[0]) @ W0[H + 4:H + 8]
    tabsP = jnp.stack([jnp.tile(e0, 8), jnp.tile(de0, 8), jnp.tile(da3, 8)])
    eye8 = jnp.eye(8, dtype=f32)
    w0p8 = jnp.kron(eye8, W0[H:H + 4])

    pd = _spmm(ones_v, zeros_v, ei_blk).reshape(NC, HP, 128)

    dinv, v = pl.pallas_call(
        _prep_body,
        out_shape=[jax.ShapeDtypeStruct((HP, 128), f32),
                   jax.ShapeDtypeStruct((HP, 128), f32)],
    )(pd, m0, m3, posP, tabsP, w0p8)

    layer_call = pl.pallas_call(
        _layer_body,
        out_shape=jax.ShapeDtypeStruct((HP, 128), f32),
    )
    biases = [b0] + [bs[j] for j in range(NL - 2)]
    for i in range(NL - 1):
        p = _spmm(v.reshape(NPAD, H), zeros_v, ei_blk).reshape(NC, HP, 128)
        v = layer_call(p, dinv, jnp.tile(biases[i], 8).reshape(1, 128),
                       jnp.kron(eye8, Ws[i]))

    p = _spmm(v.reshape(NPAD, H), zeros_v, ei_blk).reshape(NC, HP, 128)
    batchP = jnp.concatenate(
        [batch.astype(jnp.int32),
         jnp.full((NPAD - N,), NG, jnp.int32)]).reshape(HP, 8)
    out = pl.pallas_call(
        _final_body,
        out_shape=jax.ShapeDtypeStruct((NG, OUT), f32),
    )(p, dinv, batchP, bs[NL - 2].reshape(1, H), Wl, bl.reshape(1, OUT))
    return out


# speedup vs baseline: 93.7205x; 1.5097x over previous
import functools

import jax
import jax.numpy as jnp
from jax import lax
from jax.experimental import pallas as pl
from jax.experimental.pallas import tpu as pltpu
from jax.experimental.pallas import tpu_sc as plsc

N = 50000
E = 1600000
H = 16
NG = 64
NL = 15
OUT = 8

NC = 2
NS = 16
NW = NC * NS
EBW = 128
EB = 12544
EPAD = EB * EBW
BPW = EB // NW
NPAD = 50048
RPS = NPAD // NS

_mesh = plsc.VectorSubcoreMesh(core_axis_name="c", subcore_axis_name="s")


@functools.partial(
    pl.kernel,
    out_type=jax.ShapeDtypeStruct((NC, NPAD, H), jnp.float32),
    mesh=_mesh,
    scratch_types=[
        pltpu.VMEM((BPW // 4, 2, EBW), jnp.int32),
        pltpu.VMEM((BPW // 4, 2, EBW), jnp.int32),
        *[pltpu.VMEM((EBW, H), jnp.float32) for _ in range(8)],
        pltpu.VMEM_SHARED((NPAD, H), jnp.float32),
        *[pltpu.SemaphoreType.DMA for _ in range(18)],
    ],
    compiler_params=pltpu.CompilerParams(use_tc_tiling_on_sc=False),
)
def _spmm(v_hbm, zero_hbm, ei_hbm, out_hbm, idx0, idx1,
          gbuf0, gbuf1, gbuf2, gbuf3, gbuf4, gbuf5, gbuf6, gbuf7, acc,
          gsem0, gsem1, gsem2, gsem3, gsem4, gsem5, gsem6, gsem7,
          ssem0, ssem1, ssem2, ssem3, ssem4, ssem5, ssem6, ssem7,
          isem0, isem1):
    cid = lax.axis_index("c")
    sid = lax.axis_index("s")
    wid = sid * NC + cid
    rows = pl.ds(sid * RPS, RPS)
    base = wid * BPW
    qbpw = BPW // 4

    pltpu.sync_copy(ei_hbm.at[pl.ds(base, qbpw)], idx0)
    pltpu.make_async_copy(ei_hbm.at[pl.ds(base + qbpw, qbpw)], idx1,
                          isem1).start()

    @pl.when(cid == 0)
    def _():
        pltpu.sync_copy(v_hbm.at[rows], acc.at[rows])

    @pl.when(cid == 1)
    def _():
        pltpu.sync_copy(zero_hbm.at[rows], acc.at[rows])

    plsc.subcore_barrier()

    gbufs = (gbuf0, gbuf1, gbuf2, gbuf3, gbuf4, gbuf5, gbuf6, gbuf7)
    gsems = (gsem0, gsem1, gsem2, gsem3, gsem4, gsem5, gsem6, gsem7)
    ssems = (ssem0, ssem1, ssem2, ssem3, ssem4, ssem5, ssem6, ssem7)
    D = 8

    def _run_chunk(idx):
        def sg(b, r):
            pltpu.make_async_copy(v_hbm.at[idx.at[b].at[0]], gbufs[r],
                                  gsems[r]).start()

        def wg(b, r):
            pltpu.make_async_copy(v_hbm.at[idx.at[b].at[0]], gbufs[r],
                                  gsems[r]).wait()

        def sc(b, r):
            pltpu.make_async_copy(gbufs[r], acc.at[idx.at[b].at[1]],
                                  ssems[r]).start(add=True)

        def ws(r):
            pltpu.make_async_copy(gbufs[r], acc.at[idx.at[0].at[1]],
                                  ssems[r]).wait()

        for b in range(4):
            sg(b, b)
        for b in range(4):
            wg(b, b)
            sc(b, b)
            sg(b + 4, b + 4)

        @pl.loop(4, qbpw - 6, step=D)
        def _(b):
            for k in range(D):
                r = (4 + k) % D
                bb = b + k
                wg(bb, r)
                sc(bb, r)
                r2 = (r + 4) % D
                ws(r2)
                sg(bb + 4, r2)

        for bb in (qbpw - 6, qbpw - 5):
            r = bb % D
            wg(bb, r)
            sc(bb, r)
            r2 = (r + 4) % D
            ws(r2)
            sg(bb + 4, r2)
        for bb in (qbpw - 4, qbpw - 3, qbpw - 2, qbpw - 1):
            r = bb % D
            wg(bb, r)
            sc(bb, r)
        for r in range(D):
            ws(r)

    _run_chunk(idx0)
    pltpu.make_async_copy(ei_hbm.at[pl.ds(base + 2 * qbpw, qbpw)], idx0,
                          isem0).start()
    pltpu.make_async_copy(ei_hbm.at[pl.ds(base + qbpw, qbpw)], idx1,
                          isem1).wait()
    _run_chunk(idx1)
    pltpu.make_async_copy(ei_hbm.at[pl.ds(base + 3 * qbpw, qbpw)], idx1,
                          isem1).start()
    pltpu.make_async_copy(ei_hbm.at[pl.ds(base + 2 * qbpw, qbpw)], idx0,
                          isem0).wait()
    _run_chunk(idx0)
    pltpu.make_async_copy(ei_hbm.at[pl.ds(base + 3 * qbpw, qbpw)], idx1,
                          isem1).wait()
    _run_chunk(idx1)

    plsc.subcore_barrier()
    pltpu.sync_copy(acc.at[rows], out_hbm.at[cid].at[rows])


HP = NPAD // 8


def _prep_body(pd_ref, m0_ref, m3_ref, pos_ref, tabs_ref, w0p8_ref,
               dinv_ref, v_ref):
    dinv = lax.rsqrt(pd_ref[0] + pd_ref[1])
    dinv_ref[...] = dinv
    g = (tabs_ref[0:1, :] + m0_ref[...] * tabs_ref[1:2, :]
         + m3_ref[...] * tabs_ref[2:3, :])
    v_ref[...] = dinv * (g + pos_ref[...] @ w0p8_ref[...])


def _layer_body(p_ref, dinv_ref, b_ref, w8_ref, o_ref):
    h = dinv_ref[...] * (p_ref[0] + p_ref[1]) + b_ref[...]
    o_ref[...] = dinv_ref[...] * (jnp.maximum(h, 0.0) @ w8_ref[...])


def _final_body(p_ref, dinv_ref, batch_ref, b_ref, wl_ref, bl_ref, o_ref):
    h = dinv_ref[...] * (p_ref[0] + p_ref[1])
    sums = jnp.zeros((NG, H), jnp.float32)
    cnt = jnp.zeros((NG, 1), jnp.float32)
    iota = lax.broadcasted_iota(jnp.int32, (HP, NG), 1)
    for j in range(8):
        oh = (batch_ref[:, j:j + 1] == iota).astype(jnp.float32)
        sums += lax.dot_general(oh, h[:, j * H:(j + 1) * H],
                                (((0,), (0,)), ((), ())))
        cnt += jnp.sum(oh, axis=0)[:, None]
    pooled = sums / jnp.maximum(cnt, 1.0) + b_ref[...]
    o_ref[...] = pooled @ wl_ref[...] + bl_ref[...]


def kernel(x, edge_index, batch, embed, embed_agent, W0, b0, Ws, bs, Wl, bl):
    f32 = jnp.float32

    pad_idx = N + (jnp.arange(EPAD - E, dtype=jnp.int32) % (NPAD - N))
    ei = jnp.concatenate(
        [edge_index.astype(jnp.int32),
         jnp.stack([pad_idx, pad_idx])], axis=1)
    ei_blk = jnp.stack([ei[0].reshape(EB, EBW), ei[1].reshape(EB, EBW)],
                       axis=1)

    ones_v = jnp.ones((NPAD, H), f32)
    zeros_v = jnp.zeros((NPAD, H), f32)

    div_term = 1.0 / 10000.0
    xp = x[:, 1:2].astype(f32) * div_term
    yp = x[:, 2:3].astype(f32) * div_term
    xpos = jnp.concatenate([jnp.sin(xp), jnp.cos(xp)], axis=0).reshape(-1, 2)
    ypos = jnp.concatenate([jnp.sin(yp), jnp.cos(yp)], axis=0).reshape(-1, 2)
    pos4 = jnp.concatenate([xpos, ypos], axis=1)
    pos4 = jnp.concatenate([pos4, jnp.zeros((NPAD - N, 4), f32)], axis=0)
    posP = pos4.reshape(HP, 32)
    x_pad = jnp.concatenate(
        [x.astype(jnp.int32), jnp.zeros((NPAD - N, 4), jnp.int32)], axis=0)
    m0 = jnp.repeat(x_pad[:, 0].astype(f32), H).reshape(HP, 128)
    m3 = jnp.repeat(x_pad[:, 3].astype(f32), H).reshape(HP, 128)

    e0 = embed[0] @ W0[0:H] + embed_agent[0] @ W0[H + 4:H + 8]
    de0 = (embed[1] - embed[0]) @ W0[0:H]
    da3 = (embed_agent[1] - embed_agent[0]) @ W0[H + 4:H + 8]
    tabsP = jnp.stack([jnp.tile(e0, 8), jnp.tile(de0, 8), jnp.tile(da3, 8)])
    eye8 = jnp.eye(8, dtype=f32)
    w0p8 = jnp.kron(eye8, W0[H:H + 4])

    pd = _spmm(ones_v, zeros_v, ei_blk).reshape(NC, HP, 128)

    dinv, v = pl.pallas_call(
        _prep_body,
        out_shape=[jax.ShapeDtypeStruct((HP, 128), f32),
                   jax.ShapeDtypeStruct((HP, 128), f32)],
    )(pd, m0, m3, posP, tabsP, w0p8)

    layer_call = pl.pallas_call(
        _layer_body,
        out_shape=jax.ShapeDtypeStruct((HP, 128), f32),
    )
    biases = [b0] + [bs[j] for j in range(NL - 2)]
    for i in range(NL - 1):
        p = _spmm(v.reshape(NPAD, H), zeros_v, ei_blk).reshape(NC, HP, 128)
        v = layer_call(p, dinv, jnp.tile(biases[i], 8).reshape(1, 128),
                       jnp.kron(eye8, Ws[i]))

    p = _spmm(v.reshape(NPAD, H), zeros_v, ei_blk).reshape(NC, HP, 128)
    batchP = jnp.concatenate(
        [batch.astype(jnp.int32),
         jnp.full((NPAD - N,), NG, jnp.int32)]).reshape(HP, 8)
    out = pl.pallas_call(
        _final_body,
        out_shape=jax.ShapeDtypeStruct((NG, OUT), f32),
    )(p, dinv, batchP, bs[NL - 2].reshape(1, H), Wl, bl.reshape(1, OUT))
    return out


# speedup vs baseline: 114.8112x; 1.2250x over previous
import functools

import jax
import jax.numpy as jnp
from jax import lax
from jax.experimental import pallas as pl
from jax.experimental.pallas import tpu as pltpu
from jax.experimental.pallas import tpu_sc as plsc

N = 50000
E = 1600000
H = 16
NG = 64
NL = 15
OUT = 8

NC = 2
NS = 16
NW = NC * NS
EBW = 128
EB = 12544
EPAD = EB * EBW
BPW = EB // NW
NPAD = 50048
RPS = NPAD // NS

_mesh = plsc.VectorSubcoreMesh(core_axis_name="c", subcore_axis_name="s")


_D = 14


@functools.partial(
    pl.kernel,
    out_type=jax.ShapeDtypeStruct((NC, NPAD, H), jnp.float32),
    mesh=_mesh,
    scratch_types=[
        pltpu.VMEM((BPW // 4, 2, EBW), jnp.int32),
        pltpu.VMEM((BPW // 4, 2, EBW), jnp.int32),
        *[pltpu.VMEM((EBW, H), jnp.float32) for _ in range(_D)],
        pltpu.VMEM_SHARED((NPAD, H), jnp.float32),
        *[pltpu.SemaphoreType.DMA for _ in range(2 * _D + 2)],
    ],
    compiler_params=pltpu.CompilerParams(use_tc_tiling_on_sc=False),
)
def _spmm(v_hbm, zero_hbm, ei_hbm, out_hbm, idx0, idx1, *scr):
    gbufs = scr[:_D]
    acc = scr[_D]
    gsems = scr[_D + 1:2 * _D + 1]
    ssems = scr[2 * _D + 1:3 * _D + 1]
    isem0 = scr[3 * _D + 1]
    isem1 = scr[3 * _D + 2]
    cid = lax.axis_index("c")
    sid = lax.axis_index("s")
    wid = sid * NC + cid
    rows = pl.ds(sid * RPS, RPS)
    base = wid * BPW
    qbpw = BPW // 4

    pltpu.sync_copy(ei_hbm.at[pl.ds(base, qbpw)], idx0)
    pltpu.make_async_copy(ei_hbm.at[pl.ds(base + qbpw, qbpw)], idx1,
                          isem1).start()

    @pl.when(cid == 0)
    def _():
        pltpu.sync_copy(v_hbm.at[rows], acc.at[rows])

    @pl.when(cid == 1)
    def _():
        pltpu.sync_copy(zero_hbm.at[rows], acc.at[rows])

    plsc.subcore_barrier()

    D = _D
    HD = D // 2

    def _run_chunk(idx):
        def sg(b, r):
            pltpu.make_async_copy(v_hbm.at[idx.at[b].at[0]], gbufs[r],
                                  gsems[r]).start()

        def wg(b, r):
            pltpu.make_async_copy(v_hbm.at[idx.at[b].at[0]], gbufs[r],
                                  gsems[r]).wait()

        def sc(b, r):
            pltpu.make_async_copy(gbufs[r], acc.at[idx.at[b].at[1]],
                                  ssems[r]).start(add=True)

        def ws(r):
            pltpu.make_async_copy(gbufs[r], acc.at[idx.at[0].at[1]],
                                  ssems[r]).wait()

        for b in range(HD):
            sg(b, b)
        for b in range(HD):
            wg(b, b)
            sc(b, b)
            sg(b + HD, b + HD)

        @pl.loop(HD, qbpw - HD, step=D)
        def _(b):
            for k in range(D):
                r = (HD + k) % D
                bb = b + k
                wg(bb, r)
                sc(bb, r)
                r2 = (r + HD) % D
                ws(r2)
                sg(bb + HD, r2)

        for bb in range(qbpw - HD, qbpw):
            r = bb % D
            wg(bb, r)
            sc(bb, r)
        for r in range(D):
            ws(r)

    _run_chunk(idx0)
    pltpu.make_async_copy(ei_hbm.at[pl.ds(base + 2 * qbpw, qbpw)], idx0,
                          isem0).start()
    pltpu.make_async_copy(ei_hbm.at[pl.ds(base + qbpw, qbpw)], idx1,
                          isem1).wait()
    _run_chunk(idx1)
    pltpu.make_async_copy(ei_hbm.at[pl.ds(base + 3 * qbpw, qbpw)], idx1,
                          isem1).start()
    pltpu.make_async_copy(ei_hbm.at[pl.ds(base + 2 * qbpw, qbpw)], idx0,
                          isem0).wait()
    _run_chunk(idx0)
    pltpu.make_async_copy(ei_hbm.at[pl.ds(base + 3 * qbpw, qbpw)], idx1,
                          isem1).wait()
    _run_chunk(idx1)

    plsc.subcore_barrier()
    pltpu.sync_copy(acc.at[rows], out_hbm.at[cid].at[rows])


HP = NPAD // 8


def _prep_body(pd_ref, m0_ref, m3_ref, pos_ref, tabs_ref, w0p8_ref,
               dinv_ref, v_ref):
    dinv = lax.rsqrt(pd_ref[0] + pd_ref[1])
    dinv_ref[...] = dinv
    g = (tabs_ref[0:1, :] + m0_ref[...] * tabs_ref[1:2, :]
         + m3_ref[...] * tabs_ref[2:3, :])
    v_ref[...] = dinv * (g + pos_ref[...] @ w0p8_ref[...])


def _layer_body(p_ref, dinv_ref, b_ref, w8_ref, o_ref):
    h = dinv_ref[...] * (p_ref[0] + p_ref[1]) + b_ref[...]
    o_ref[...] = dinv_ref[...] * (jnp.maximum(h, 0.0) @ w8_ref[...])


def _final_body(p_ref, dinv_ref, batch_ref, b_ref, wl_ref, bl_ref, o_ref):
    h = dinv_ref[...] * (p_ref[0] + p_ref[1])
    sums = jnp.zeros((NG, H), jnp.float32)
    cnt = jnp.zeros((NG, 1), jnp.float32)
    iota = lax.broadcasted_iota(jnp.int32, (HP, NG), 1)
    for j in range(8):
        oh = (batch_ref[:, j:j + 1] == iota).astype(jnp.float32)
        sums += lax.dot_general(oh, h[:, j * H:(j + 1) * H],
                                (((0,), (0,)), ((), ())))
        cnt += jnp.sum(oh, axis=0)[:, None]
    pooled = sums / jnp.maximum(cnt, 1.0) + b_ref[...]
    o_ref[...] = pooled @ wl_ref[...] + bl_ref[...]


def kernel(x, edge_index, batch, embed, embed_agent, W0, b0, Ws, bs, Wl, bl):
    f32 = jnp.float32

    pad_idx = N + (jnp.arange(EPAD - E, dtype=jnp.int32) % (NPAD - N))
    ei = jnp.concatenate(
        [edge_index.astype(jnp.int32),
         jnp.stack([pad_idx, pad_idx])], axis=1)
    ei_blk = jnp.stack([ei[0].reshape(EB, EBW), ei[1].reshape(EB, EBW)],
                       axis=1)

    ones_v = jnp.ones((NPAD, H), f32)
    zeros_v = jnp.zeros((NPAD, H), f32)

    div_term = 1.0 / 10000.0
    xp = x[:, 1:2].astype(f32) * div_term
    yp = x[:, 2:3].astype(f32) * div_term
    xpos = jnp.concatenate([jnp.sin(xp), jnp.cos(xp)], axis=0).reshape(-1, 2)
    ypos = jnp.concatenate([jnp.sin(yp), jnp.cos(yp)], axis=0).reshape(-1, 2)
    pos4 = jnp.concatenate([xpos, ypos], axis=1)
    pos4 = jnp.concatenate([pos4, jnp.zeros((NPAD - N, 4), f32)], axis=0)
    posP = pos4.reshape(HP, 32)
    x_pad = jnp.concatenate(
        [x.astype(jnp.int32), jnp.zeros((NPAD - N, 4), jnp.int32)], axis=0)
    m0 = jnp.repeat(x_pad[:, 0].astype(f32), H).reshape(HP, 128)
    m3 = jnp.repeat(x_pad[:, 3].astype(f32), H).reshape(HP, 128)

    e0 = embed[0] @ W0[0:H] + embed_agent[0] @ W0[H + 4:H + 8]
    de0 = (embed[1] - embed[0]) @ W0[0:H]
    da3 = (embed_agent[1] - embed_agent[0]) @ W0[H + 4:H + 8]
    tabsP = jnp.stack([jnp.tile(e0, 8), jnp.tile(de0, 8), jnp.tile(da3, 8)])
    eye8 = jnp.eye(8, dtype=f32)
    w0p8 = jnp.kron(eye8, W0[H:H + 4])

    pd = _spmm(ones_v, zeros_v, ei_blk).reshape(NC, HP, 128)

    dinv, v = pl.pallas_call(
        _prep_body,
        out_shape=[jax.ShapeDtypeStruct((HP, 128), f32),
                   jax.ShapeDtypeStruct((HP, 128), f32)],
    )(pd, m0, m3, posP, tabsP, w0p8)

    layer_call = pl.pallas_call(
        _layer_body,
        out_shape=jax.ShapeDtypeStruct((HP, 128), f32),
    )
    biases = [b0] + [bs[j] for j in range(NL - 2)]
    for i in range(NL - 1):
        p = _spmm(v.reshape(NPAD, H), zeros_v, ei_blk).reshape(NC, HP, 128)
        v = layer_call(p, dinv, jnp.tile(biases[i], 8).reshape(1, 128),
                       jnp.kron(eye8, Ws[i]))

    p = _spmm(v.reshape(NPAD, H), zeros_v, ei_blk).reshape(NC, HP, 128)
    batchP = jnp.concatenate(
        [batch.astype(jnp.int32),
         jnp.full((NPAD - N,), NG, jnp.int32)]).reshape(HP, 8)
    out = pl.pallas_call(
        _final_body,
        out_shape=jax.ShapeDtypeStruct((NG, OUT), f32),
    )(p, dinv, batchP, bs[NL - 2].reshape(1, H), Wl, bl.reshape(1, OUT))
    return out


# speedup vs baseline: 116.4761x; 1.0145x over previous
import functools

import jax
import jax.numpy as jnp
from jax import lax
from jax.experimental import pallas as pl
from jax.experimental.pallas import tpu as pltpu
from jax.experimental.pallas import tpu_sc as plsc

N = 50000
E = 1600000
H = 16
NG = 64
NL = 15
OUT = 8

NC = 2
NS = 16
NW = NC * NS
EBW = 128
EB = 12544
EPAD = EB * EBW
BPW = EB // NW
NPAD = 50048
RPS = NPAD // NS

_mesh = plsc.VectorSubcoreMesh(core_axis_name="c", subcore_axis_name="s")


_D = 14


@functools.partial(
    pl.kernel,
    out_type=jax.ShapeDtypeStruct((NC, NPAD, H), jnp.float32),
    mesh=_mesh,
    scratch_types=[
        pltpu.VMEM((2, BPW // 4, EBW), jnp.int32),
        pltpu.VMEM((2, BPW // 4, EBW), jnp.int32),
        *[pltpu.VMEM((EBW, H), jnp.float32) for _ in range(_D)],
        pltpu.VMEM_SHARED((NPAD, H), jnp.float32),
        *[pltpu.SemaphoreType.DMA for _ in range(2 * _D + 2)],
    ],
    compiler_params=pltpu.CompilerParams(use_tc_tiling_on_sc=False),
)
def _spmm(v_hbm, zero_hbm, ei_hbm, out_hbm, idx0, idx1, *scr):
    gbufs = scr[:_D]
    acc = scr[_D]
    gsems = scr[_D + 1:2 * _D + 1]
    ssems = scr[2 * _D + 1:3 * _D + 1]
    isem0 = scr[3 * _D + 1]
    isem1 = scr[3 * _D + 2]
    cid = lax.axis_index("c")
    sid = lax.axis_index("s")
    wid = sid * NC + cid
    rows = pl.ds(sid * RPS, RPS)
    base = wid * BPW
    qbpw = BPW // 4

    def iload(q, buf, sem):
        c0 = pltpu.make_async_copy(ei_hbm.at[0].at[pl.ds(base + q * qbpw,
                                                         qbpw)],
                                   buf.at[0], sem)
        c1 = pltpu.make_async_copy(ei_hbm.at[1].at[pl.ds(base + q * qbpw,
                                                         qbpw)],
                                   buf.at[1], sem)
        return (c0, c1)

    def istart(q, buf, sem):
        for c in iload(q, buf, sem):
            c.start()

    def iwait(q, buf, sem):
        for c in iload(q, buf, sem):
            c.wait()

    istart(0, idx0, isem0)
    iwait(0, idx0, isem0)
    istart(1, idx1, isem1)

    @pl.when(cid == 0)
    def _():
        pltpu.sync_copy(v_hbm.at[rows], acc.at[rows])

    @pl.when(cid == 1)
    def _():
        pltpu.sync_copy(zero_hbm.at[rows], acc.at[rows])

    plsc.subcore_barrier()

    D = _D
    HD = D // 2

    def _run_chunk(idx):
        def sg(b, r):
            pltpu.make_async_copy(v_hbm.at[idx.at[0].at[b]], gbufs[r],
                                  gsems[r]).start()

        def wg(b, r):
            pltpu.make_async_copy(v_hbm.at[idx.at[0].at[b]], gbufs[r],
                                  gsems[r]).wait()

        def sc(b, r):
            pltpu.make_async_copy(gbufs[r], acc.at[idx.at[1].at[b]],
                                  ssems[r]).start(add=True)

        def ws(r):
            pltpu.make_async_copy(gbufs[r], acc.at[idx.at[1].at[0]],
                                  ssems[r]).wait()

        for b in range(HD):
            sg(b, b)
        for b in range(HD):
            wg(b, b)
            sc(b, b)
            sg(b + HD, b + HD)

        @pl.loop(HD, qbpw - HD, step=D)
        def _(b):
            for k in range(D):
                r = (HD + k) % D
                bb = b + k
                wg(bb, r)
                sc(bb, r)
                r2 = (r + HD) % D
                ws(r2)
                sg(bb + HD, r2)

        for bb in range(qbpw - HD, qbpw):
            r = bb % D
            wg(bb, r)
            sc(bb, r)
        for r in range(D):
            ws(r)

    _run_chunk(idx0)
    istart(2, idx0, isem0)
    iwait(1, idx1, isem1)
    _run_chunk(idx1)
    istart(3, idx1, isem1)
    iwait(2, idx0, isem0)
    _run_chunk(idx0)
    iwait(3, idx1, isem1)
    _run_chunk(idx1)

    plsc.subcore_barrier()
    pltpu.sync_copy(acc.at[rows], out_hbm.at[cid].at[rows])


@functools.partial(
    pl.kernel,
    out_type=jax.ShapeDtypeStruct((NC, NPAD, H), jnp.float32),
    mesh=_mesh,
    scratch_types=[
        pltpu.VMEM((BPW // 4, EBW), jnp.int32),
        pltpu.VMEM((BPW // 4, EBW), jnp.int32),
        pltpu.VMEM((EBW, H), jnp.float32),
        pltpu.VMEM_SHARED((NPAD, H), jnp.float32),
        *[pltpu.SemaphoreType.DMA for _ in range(_D + 2)],
    ],
    compiler_params=pltpu.CompilerParams(use_tc_tiling_on_sc=False),
)
def _degk(zero_hbm, ones_hbm, ei_hbm, out_hbm, idx0, idx1, obuf, acc, *sems):
    ssems = sems[:_D]
    isem0 = sems[_D]
    isem1 = sems[_D + 1]
    cid = lax.axis_index("c")
    sid = lax.axis_index("s")
    wid = sid * NC + cid
    rows = pl.ds(sid * RPS, RPS)
    base = wid * BPW
    qbpw = BPW // 4

    pltpu.sync_copy(ones_hbm, obuf)

    def istart(q, buf, sem):
        pltpu.make_async_copy(
            ei_hbm.at[1].at[pl.ds(base + q * qbpw, qbpw)], buf, sem).start()

    def iwait(q, buf, sem):
        pltpu.make_async_copy(
            ei_hbm.at[1].at[pl.ds(base + q * qbpw, qbpw)], buf, sem).wait()

    istart(0, idx0, isem0)
    iwait(0, idx0, isem0)
    istart(1, idx1, isem1)
    pltpu.sync_copy(zero_hbm.at[rows], acc.at[rows])
    plsc.subcore_barrier()

    def _run_chunk(idx):
        def sc(b, r):
            pltpu.make_async_copy(obuf, acc.at[idx.at[b]],
                                  ssems[r]).start(add=True)

        def ws(r):
            pltpu.make_async_copy(obuf, acc.at[idx.at[0]],
                                  ssems[r]).wait()

        for b in range(_D):
            sc(b, b)

        @pl.loop(_D, qbpw, step=_D)
        def _(b):
            for k in range(_D):
                ws(k)
                sc(b + k, k)

        for r in range(_D):
            ws(r)

    _run_chunk(idx0)
    istart(2, idx0, isem0)
    iwait(1, idx1, isem1)
    _run_chunk(idx1)
    istart(3, idx1, isem1)
    iwait(2, idx0, isem0)
    _run_chunk(idx0)
    iwait(3, idx1, isem1)
    _run_chunk(idx1)

    plsc.subcore_barrier()
    pltpu.sync_copy(acc.at[rows], out_hbm.at[cid].at[rows])


HP = NPAD // 8


def _prep_body(pd_ref, m0_ref, m3_ref, pos_ref, tabs_ref, w0p8_ref,
               dinv_ref, v_ref):
    dinv = lax.rsqrt(pd_ref[0] + pd_ref[1] + 1.0)
    dinv_ref[...] = dinv
    g = (tabs_ref[0:1, :] + m0_ref[...] * tabs_ref[1:2, :]
         + m3_ref[...] * tabs_ref[2:3, :])
    v_ref[...] = dinv * (g + pos_ref[...] @ w0p8_ref[...])


def _layer_body(p_ref, dinv_ref, b_ref, w8_ref, o_ref):
    h = dinv_ref[...] * (p_ref[0] + p_ref[1]) + b_ref[...]
    o_ref[...] = dinv_ref[...] * (jnp.maximum(h, 0.0) @ w8_ref[...])


def _final_body(p_ref, dinv_ref, batch_ref, b_ref, wl_ref, bl_ref, o_ref):
    h = dinv_ref[...] * (p_ref[0] + p_ref[1])
    sums = jnp.zeros((NG, H), jnp.float32)
    cnt = jnp.zeros((NG, 1), jnp.float32)
    iota = lax.broadcasted_iota(jnp.int32, (HP, NG), 1)
    for j in range(8):
        oh = (batch_ref[:, j:j + 1] == iota).astype(jnp.float32)
        sums += lax.dot_general(oh, h[:, j * H:(j + 1) * H],
                                (((0,), (0,)), ((), ())))
        cnt += jnp.sum(oh, axis=0)[:, None]
    pooled = sums / jnp.maximum(cnt, 1.0) + b_ref[...]
    o_ref[...] = pooled @ wl_ref[...] + bl_ref[...]


def kernel(x, edge_index, batch, embed, embed_agent, W0, b0, Ws, bs, Wl, bl):
    f32 = jnp.float32

    pad_idx = N + (jnp.arange(EPAD - E, dtype=jnp.int32) % (NPAD - N))
    ei_blk = jnp.concatenate(
        [edge_index.astype(jnp.int32),
         jnp.stack([pad_idx, pad_idx])], axis=1).reshape(2, EB, EBW)

    zeros_v = jnp.zeros((NPAD, H), f32)

    div_term = 1.0 / 10000.0
    xp = x[:, 1:2].astype(f32) * div_term
    yp = x[:, 2:3].astype(f32) * div_term
    xpos = jnp.concatenate([jnp.sin(xp), jnp.cos(xp)], axis=0).reshape(-1, 2)
    ypos = jnp.concatenate([jnp.sin(yp), jnp.cos(yp)], axis=0).reshape(-1, 2)
    pos4 = jnp.concatenate([xpos, ypos], axis=1)
    pos4 = jnp.concatenate([pos4, jnp.zeros((NPAD - N, 4), f32)], axis=0)
    posP = pos4.reshape(HP, 32)
    x_pad = jnp.concatenate(
        [x.astype(jnp.int32), jnp.zeros((NPAD - N, 4), jnp.int32)], axis=0)
    m0 = jnp.repeat(x_pad[:, 0].astype(f32), H).reshape(HP, 128)
    m3 = jnp.repeat(x_pad[:, 3].astype(f32), H).reshape(HP, 128)

    e0 = embed[0] @ W0[0:H] + embed_agent[0] @ W0[H + 4:H + 8]
    de0 = (embed[1] - embed[0]) @ W0[0:H]
    da3 = (embed_agent[1] - embed_agent[0]) @ W0[H + 4:H + 8]
    tabsP = jnp.stack([jnp.tile(e0, 8), jnp.tile(de0, 8), jnp.tile(da3, 8)])
    eye8 = jnp.eye(8, dtype=f32)
    w0p8 = jnp.kron(eye8, W0[H:H + 4])

    ones_blk = jnp.ones((EBW, H), f32)
    pd = _degk(zeros_v, ones_blk, ei_blk).reshape(NC, HP, 128)

    dinv, v = pl.pallas_call(
        _prep_body,
        out_shape=[jax.ShapeDtypeStruct((HP, 128), f32),
                   jax.ShapeDtypeStruct((HP, 128), f32)],
    )(pd, m0, m3, posP, tabsP, w0p8)

    layer_call = pl.pallas_call(
        _layer_body,
        out_shape=jax.ShapeDtypeStruct((HP, 128), f32),
    )
    biases = [b0] + [bs[j] for j in range(NL - 2)]
    for i in range(NL - 1):
        p = _spmm(v.reshape(NPAD, H), zeros_v, ei_blk).reshape(NC, HP, 128)
        v = layer_call(p, dinv, jnp.tile(biases[i], 8).reshape(1, 128),
                       jnp.kron(eye8, Ws[i]))

    p = _spmm(v.reshape(NPAD, H), zeros_v, ei_blk).reshape(NC, HP, 128)
    batchP = jnp.concatenate(
        [batch.astype(jnp.int32),
         jnp.full((NPAD - N,), NG, jnp.int32)]).reshape(HP, 8)
    out = pl.pallas_call(
        _final_body,
        out_shape=jax.ShapeDtypeStruct((NG, OUT), f32),
    )(p, dinv, batchP, bs[NL - 2].reshape(1, H), Wl, bl.reshape(1, OUT))
    return out
